# Initial kernel scaffold; baseline (speedup 1.0000x reference)
import functools

import jax
import jax.numpy as jnp
from jax import lax
from jax.experimental import pallas as pl
from jax.experimental.pallas import tpu as pltpu
from jax.experimental.pallas import tpu_sc as plsc

N_NODES_C = 10000
N_EDGES_C = 320000
D = 128
LANES = 16


def _zmat_body(x_ref, t_ref, o_ref):
    o_ref[...] = lax.dot_general(
        x_ref[...],
        t_ref[...],
        (((1,), (1,)), ((), ())),
        preferred_element_type=jnp.float32,
        precision=lax.Precision.HIGHEST,
    )


def _z_matmul(x, t0):
    n = x.shape[0]
    blk = 2000
    return pl.pallas_call(
        _zmat_body,
        out_shape=jax.ShapeDtypeStruct((n, D), jnp.float32),
        grid=(n // blk,),
        in_specs=[
            pl.BlockSpec((blk, D), lambda i: (i, 0)),
            pl.BlockSpec((D, D), lambda i: (0, 0)),
        ],
        out_specs=pl.BlockSpec((blk, D), lambda i: (i, 0)),
    )(x, t0)


_INFO = plsc.get_sparse_core_info()
_NC = _INFO.num_cores
_NS = _INFO.num_subcores
_NW = _NC * _NS
_EPW = N_EDGES_C // _NW
_CHUNK = 400
_NCHUNK = _EPW // _CHUNK
_GROUPS = _CHUNK // LANES


def _edge_dot_body(x_hbm, z_hbm, row_hbm, col_hbm, out_hbm,
                   ridx_v, cidx_v, xr_v, zc_v, out_v, sem_x, sem_z):
    wid = lax.axis_index("s") * _NC + lax.axis_index("c")
    wbase = wid * _EPW
    lane = lax.iota(jnp.int32, LANES)

    def chunk_body(c, carry):
        base = pl.multiple_of(wbase + c * _CHUNK, 8)
        pltpu.sync_copy(row_hbm.at[pl.ds(base, _CHUNK)], ridx_v)
        pltpu.sync_copy(col_hbm.at[pl.ds(base, _CHUNK)], cidx_v)
        cp_x = pltpu.async_copy(x_hbm.at[ridx_v], xr_v, sem_x)
        cp_z = pltpu.async_copy(z_hbm.at[cidx_v], zc_v, sem_z)
        cp_x.wait()
        cp_z.wait()

        def group_body(g, carry2):
            e_vec = pl.multiple_of(g * LANES, LANES) + lane
            accs = [jnp.zeros((LANES,), jnp.float32) for _ in range(4)]
            for d in range(D):
                dv = jnp.full((LANES,), d, jnp.int32)
                xv = plsc.load_gather(xr_v, [e_vec, dv])
                zv = plsc.load_gather(zc_v, [e_vec, dv])
                accs[d % 4] = accs[d % 4] + xv * zv
            m = (accs[0] + accs[1]) + (accs[2] + accs[3])
            t = jnp.exp(jnp.abs(m) * -2.0)
            r = (1.0 - t) / (1.0 + t)
            out_v[pl.ds(pl.multiple_of(g * LANES, LANES), LANES)] = (
                jnp.where(m < 0.0, -r, r))
            return carry2

        lax.fori_loop(0, _GROUPS, group_body, 0, unroll=False)
        pltpu.sync_copy(out_v, out_hbm.at[pl.ds(base, _CHUNK)])
        return carry

    lax.fori_loop(0, _NCHUNK, chunk_body, 0, unroll=False)


def _edge_dot(x, z, row, col):
    mesh = plsc.VectorSubcoreMesh(core_axis_name="c", subcore_axis_name="s")
    kern = functools.partial(
        pl.kernel,
        mesh=mesh,
        out_type=jax.ShapeDtypeStruct((N_EDGES_C,), jnp.float32),
        scratch_types=[
            pltpu.VMEM((_CHUNK,), jnp.int32),
            pltpu.VMEM((_CHUNK,), jnp.int32),
            pltpu.VMEM((_CHUNK, D), jnp.float32),
            pltpu.VMEM((_CHUNK, D), jnp.float32),
            pltpu.VMEM((_CHUNK,), jnp.float32),
            pltpu.SemaphoreType.DMA,
            pltpu.SemaphoreType.DMA,
        ],
    )(_edge_dot_body)
    return kern(x, z, row, col)


def kernel(x, edge_index, tensor):
    row = edge_index[0].astype(jnp.int32)
    col = edge_index[1].astype(jnp.int32)
    z = _z_matmul(x, tensor[0])
    maps = _edge_dot(x, z, row, col)
    return maps.reshape(-1, 1)


# speedup vs baseline: 3.1469x; 3.1469x over previous
import functools

import jax
import jax.numpy as jnp
from jax import lax
from jax.experimental import pallas as pl
from jax.experimental.pallas import tpu as pltpu
from jax.experimental.pallas import tpu_sc as plsc

N_NODES_C = 10000
N_EDGES_C = 320000
D = 128
LANES = 16


def _zmat_body(x_ref, t_ref, o_ref):
    o_ref[...] = lax.dot_general(
        x_ref[...],
        t_ref[...],
        (((1,), (1,)), ((), ())),
        preferred_element_type=jnp.float32,
        precision=lax.Precision.HIGHEST,
    )


def _z_matmul(x, t0):
    n = x.shape[0]
    blk = 2000
    return pl.pallas_call(
        _zmat_body,
        out_shape=jax.ShapeDtypeStruct((n, D), jnp.float32),
        grid=(n // blk,),
        in_specs=[
            pl.BlockSpec((blk, D), lambda i: (i, 0)),
            pl.BlockSpec((D, D), lambda i: (0, 0)),
        ],
        out_specs=pl.BlockSpec((blk, D), lambda i: (i, 0)),
    )(x, t0)


_INFO = plsc.get_sparse_core_info()
_NC = _INFO.num_cores
_NS = _INFO.num_subcores
_NW = _NC * _NS
_EPW = N_EDGES_C // _NW
_CHUNK = 400
_NCHUNK = _EPW // _CHUNK
_GROUPS = _CHUNK // LANES


def _edge_dot_body(x_hbm, z_hbm, row_hbm, col_hbm, out_hbm,
                   ridx_v, cidx_v, xr_v, zc_v, out_v, sem_x, sem_z):
    wid = lax.axis_index("s") * _NC + lax.axis_index("c")
    wbase = wid * _EPW
    lane = lax.iota(jnp.int32, LANES)

    def chunk_body(c, carry):
        base = pl.multiple_of(wbase + c * _CHUNK, 8)
        pltpu.sync_copy(row_hbm.at[pl.ds(base, _CHUNK)], ridx_v)
        pltpu.sync_copy(col_hbm.at[pl.ds(base, _CHUNK)], cidx_v)
        cp_x = pltpu.async_copy(x_hbm.at[ridx_v], xr_v, sem_x)
        cp_z = pltpu.async_copy(z_hbm.at[cidx_v], zc_v, sem_z)
        cp_x.wait()
        cp_z.wait()

        def group_body(g, carry2):
            m = jnp.zeros((LANES,), jnp.float32)
            for j in range(LANES):
                e = g * LANES + j
                acc = xr_v[e, pl.ds(0, LANES)] * zc_v[e, pl.ds(0, LANES)]
                for k in range(1, D // LANES):
                    acc = acc + (xr_v[e, pl.ds(k * LANES, LANES)]
                                 * zc_v[e, pl.ds(k * LANES, LANES)])
                s = jnp.sum(acc)
                m = jnp.where(lane == j, s, m)
            t = jnp.exp(jnp.abs(m) * -2.0)
            r = (1.0 - t) / (1.0 + t)
            out_v[pl.ds(pl.multiple_of(g * LANES, LANES), LANES)] = (
                jnp.where(m < 0.0, -r, r))
            return carry2

        lax.fori_loop(0, _GROUPS, group_body, 0, unroll=False)
        pltpu.sync_copy(out_v, out_hbm.at[pl.ds(base, _CHUNK)])
        return carry

    lax.fori_loop(0, _NCHUNK, chunk_body, 0, unroll=False)


def _edge_dot(x, z, row, col):
    mesh = plsc.VectorSubcoreMesh(core_axis_name="c", subcore_axis_name="s")
    kern = functools.partial(
        pl.kernel,
        mesh=mesh,
        compiler_params=pltpu.CompilerParams(needs_layout_passes=False),
        out_type=jax.ShapeDtypeStruct((N_EDGES_C,), jnp.float32),
        scratch_types=[
            pltpu.VMEM((_CHUNK,), jnp.int32),
            pltpu.VMEM((_CHUNK,), jnp.int32),
            pltpu.VMEM((_CHUNK, D), jnp.float32),
            pltpu.VMEM((_CHUNK, D), jnp.float32),
            pltpu.VMEM((_CHUNK,), jnp.float32),
            pltpu.SemaphoreType.DMA,
            pltpu.SemaphoreType.DMA,
        ],
    )(_edge_dot_body)
    return kern(x, z, row, col)


def kernel(x, edge_index, tensor):
    row = edge_index[0].astype(jnp.int32)
    col = edge_index[1].astype(jnp.int32)
    z = _z_matmul(x, tensor[0])
    maps = _edge_dot(x, z, row, col)
    return maps.reshape(-1, 1)
